# baseline (speedup 1.0000x reference)
import jax
import jax.numpy as jnp
from jax import lax
from jax.experimental import pallas as pl
from jax.experimental.pallas import tpu as pltpu
from jax.experimental.pallas import tpu_sc as plsc

N_NODES = 100000
NPAD = 100352
RPT = NPAD // 16
F = 16
M = NPAD // 8
MR = N_NODES // 8

NW = 32
CHUNK = 128
KC = 4
NGRP = 196
EPW = NGRP * KC * CHUNK
EPAD = NW * EPW
ECH = EPAD // CHUNK

BR = 448
NG = M // BR


def _make_edge_aggregate(with_cnt):
  mesh = plsc.VectorSubcoreMesh(core_axis_name="c", subcore_axis_name="s",
                                num_cores=2, num_subcores=16)

  out_type = [jax.ShapeDtypeStruct((2, NPAD, F), jnp.float32)]
  scratch = [
      pltpu.VMEM_SHARED((NPAD, F), jnp.float32),
      pltpu.VMEM((2, KC, CHUNK), jnp.int32),
      pltpu.VMEM((2, KC, CHUNK), jnp.int32),
      pltpu.VMEM((2, KC, CHUNK, F), jnp.float32),
      pltpu.SemaphoreType.DMA,
      pltpu.SemaphoreType.DMA,
      pltpu.SemaphoreType.DMA,
      pltpu.SemaphoreType.DMA,
  ]
  if with_cnt:
    out_type.append(jax.ShapeDtypeStruct((2, NPAD), jnp.float32))
    scratch += [
        pltpu.VMEM_SHARED((NPAD,), jnp.float32),
        pltpu.VMEM((CHUNK,), jnp.float32),
    ]

  def body(*refs):
    if with_cnt:
      (y_hbm, src_hbm, dst_hbm, out_hbm, cnt_hbm,
       acc_sh, sidx, didx, rows, gs0, gs1, ss0, ss1, cnt_sh, ones_v) = refs
    else:
      (y_hbm, src_hbm, dst_hbm, out_hbm,
       acc_sh, sidx, didx, rows, gs0, gs1, ss0, ss1) = refs
    gsem = (gs0, gs1)
    ssem = (ss0, ss1)

    c = lax.axis_index("c")
    s = lax.axis_index("s")
    w = s * 2 + c
    row0 = s * RPT
    base = w * NGRP * KC

    def stage(p, g):
      pltpu.sync_copy(src_hbm.at[pl.ds(base + g * KC, KC)], sidx.at[p])
      pltpu.sync_copy(dst_hbm.at[pl.ds(base + g * KC, KC)], didx.at[p])

    def fire_g(p):
      for b in range(KC):
        pltpu.async_copy(y_hbm.at[sidx.at[p, b]], rows.at[p, b], gsem[p])

    def wait_g(p):
      for b in range(KC):
        pltpu.make_async_copy(y_hbm.at[sidx.at[p, b]], rows.at[p, b],
                              gsem[p]).wait()

    def fire_s(p):
      for b in range(KC):
        pltpu.async_copy(rows.at[p, b], acc_sh.at[didx.at[p, b]], ssem[p],
                         add=True)
      if with_cnt:
        for b in range(KC):
          pltpu.async_copy(ones_v, cnt_sh.at[didx.at[p, b]], ssem[p],
                           add=True)

    def wait_s(p):
      for b in range(KC):
        pltpu.make_async_copy(rows.at[p, b], acc_sh.at[didx.at[p, b]],
                              ssem[p]).wait()
      if with_cnt:
        for b in range(KC):
          pltpu.make_async_copy(ones_v, cnt_sh.at[didx.at[p, b]],
                                ssem[p]).wait()

    def zrow(j, carry):
      rows[0, 0, j, :] = jnp.zeros((F,), jnp.float32)
      return carry
    lax.fori_loop(0, CHUNK, zrow, 0)
    for tt in range(RPT // CHUNK):
      pltpu.sync_copy(rows.at[0, 0],
                      acc_sh.at[pl.ds(row0 + tt * CHUNK, CHUNK)])
    if with_cnt:
      for i in range(CHUNK // 16):
        ones_v[pl.ds(i * 16, 16)] = jnp.zeros((16,), jnp.float32)
      for tt in range(RPT // CHUNK):
        pltpu.sync_copy(ones_v, cnt_sh.at[pl.ds(row0 + tt * CHUNK, CHUNK)])
      for i in range(CHUNK // 16):
        ones_v[pl.ds(i * 16, 16)] = jnp.full((16,), 1.0, jnp.float32)
    plsc.subcore_barrier()

    stage(0, 0)
    fire_g(0)

    def group2(i, carry):
      g = 2 * i
      stage(1, g + 1)
      fire_g(1)
      wait_g(0)
      fire_s(0)
      wait_s(0)

      @pl.when(i < NGRP // 2 - 1)
      def _():
        stage(0, g + 2)
        fire_g(0)

      wait_g(1)
      fire_s(1)
      wait_s(1)
      return carry

    lax.fori_loop(0, NGRP // 2, group2, 0)
    plsc.subcore_barrier()

    pltpu.sync_copy(acc_sh.at[pl.ds(row0, RPT)],
                    out_hbm.at[c, pl.ds(row0, RPT)])
    if with_cnt:
      pltpu.sync_copy(cnt_sh.at[pl.ds(row0, RPT)],
                      cnt_hbm.at[c, pl.ds(row0, RPT)])

  return pl.kernel(body,
                   out_type=tuple(out_type) if with_cnt else out_type[0],
                   mesh=mesh, scratch_types=scratch,
                   compiler_params=pltpu.CompilerParams(
                       use_tc_tiling_on_sc=False))


_edge_agg_cnt = _make_edge_aggregate(True)
_edge_agg = _make_edge_aggregate(False)


def _msg_body(x2_ref, b1_ref, bb1_ref, o_ref):
  o_ref[...] = jnp.maximum(
      jnp.dot(x2_ref[...], b1_ref[...], preferred_element_type=jnp.float32,
              precision=lax.Precision.HIGHEST)
      + bb1_ref[...], 0.0)


def _msg_table(x2, B1, b1r):
  return pl.pallas_call(
      _msg_body,
      grid=(NG,),
      in_specs=[pl.BlockSpec((BR, 24), lambda i: (i, 0)),
                pl.BlockSpec((24, 128), lambda i: (0, 0)),
                pl.BlockSpec((1, 128), lambda i: (0, 0))],
      out_specs=pl.BlockSpec((BR, 128), lambda i: (i, 0)),
      out_shape=jax.ShapeDtypeStruct((M, 128), jnp.float32),
  )(x2, B1, b1r)


def _upd1_body(x2_ref, s_ref, c_ref, r8_ref, kx_ref, km_ref, ba_ref,
               gm_ref, k2_ref, b2_ref, h_ref, y2_ref):
  rep = jnp.dot(c_ref[0] + c_ref[1], r8_ref[...],
                preferred_element_type=jnp.float32,
              precision=lax.Precision.HIGHEST)
  mean = (s_ref[0] + s_ref[1]) / jnp.maximum(rep, 1.0)
  h = (jnp.dot(x2_ref[...], kx_ref[...], preferred_element_type=jnp.float32,
              precision=lax.Precision.HIGHEST)
       + jnp.dot(mean, km_ref[...], preferred_element_type=jnp.float32,
              precision=lax.Precision.HIGHEST)
       + ba_ref[...])
  h = jnp.maximum(h, 0.0)
  n2 = jnp.dot(h * h, gm_ref[...], preferred_element_type=jnp.float32,
              precision=lax.Precision.HIGHEST)
  h = h / jnp.maximum(jnp.sqrt(n2), 1e-12)
  h_ref[...] = h
  y2_ref[...] = jnp.maximum(
      jnp.dot(h, k2_ref[...], preferred_element_type=jnp.float32,
              precision=lax.Precision.HIGHEST)
      + b2_ref[...], 0.0)


def _update1(x2, s1p, c8, R8, K1x, K1m, b1ar, GM, K2l, b2lr):
  full = lambda a, b: pl.BlockSpec((a, b), lambda i: (0, 0))
  return pl.pallas_call(
      _upd1_body,
      grid=(NG,),
      in_specs=[pl.BlockSpec((BR, 24), lambda i: (i, 0)),
                pl.BlockSpec((2, BR, 128), lambda i: (0, i, 0)),
                pl.BlockSpec((2, BR, 8), lambda i: (0, i, 0)),
                full(8, 128), full(24, 128), full(128, 128), full(1, 128),
                full(128, 128), full(128, 128), full(1, 128)],
      out_specs=(pl.BlockSpec((BR, 128), lambda i: (i, 0)),
                 pl.BlockSpec((BR, 128), lambda i: (i, 0))),
      out_shape=(jax.ShapeDtypeStruct((M, 128), jnp.float32),
                 jax.ShapeDtypeStruct((M, 128), jnp.float32)),
  )(x2, s1p, c8, R8, K1x, K1m, b1ar, GM, K2l, b2lr)


def _upd2_body(h1_ref, s_ref, c_ref, r8_ref, kx_ref, km_ref, ba_ref,
               gm_ref, wo_ref, gs_ref, bo_ref, o8_ref):
  rep = jnp.dot(c_ref[0] + c_ref[1], r8_ref[...],
                preferred_element_type=jnp.float32,
              precision=lax.Precision.HIGHEST)
  mean = (s_ref[0] + s_ref[1]) / jnp.maximum(rep, 1.0)
  h = (jnp.dot(h1_ref[...], kx_ref[...], preferred_element_type=jnp.float32,
              precision=lax.Precision.HIGHEST)
       + jnp.dot(mean, km_ref[...], preferred_element_type=jnp.float32,
              precision=lax.Precision.HIGHEST)
       + ba_ref[...])
  h = jnp.maximum(h, 0.0)
  n2 = jnp.dot(h * h, gm_ref[...], preferred_element_type=jnp.float32,
              precision=lax.Precision.HIGHEST)
  h = h / jnp.maximum(jnp.sqrt(n2), 1e-12)
  o8_ref[...] = jnp.dot(h * wo_ref[...], gs_ref[...],
                        preferred_element_type=jnp.float32,
              precision=lax.Precision.HIGHEST) + bo_ref[...]


def _update2(h1, s2p, c8, R8, K2x, K2m, b2ar, GM, worep, GS, bo8):
  full = lambda a, b: pl.BlockSpec((a, b), lambda i: (0, 0))
  return pl.pallas_call(
      _upd2_body,
      grid=(NG,),
      in_specs=[pl.BlockSpec((BR, 128), lambda i: (i, 0)),
                pl.BlockSpec((2, BR, 128), lambda i: (0, i, 0)),
                pl.BlockSpec((2, BR, 8), lambda i: (0, i, 0)),
                full(8, 128), full(128, 128), full(128, 128), full(1, 128),
                full(128, 128), full(1, 128), full(128, 8), full(1, 8)],
      out_specs=pl.BlockSpec((BR, 8), lambda i: (i, 0)),
      out_shape=jax.ShapeDtypeStruct((M, 8), jnp.float32),
  )(h1, s2p, c8, R8, K2x, K2m, b2ar, GM, worep, GS, bo8)


def kernel(x, edge_index, W1_lin, b1_lin, W1_agg, b1_agg,
           W2_lin, b2_lin, W2_agg, b2_agg, W_out, b_out):
  n = x.shape[0]
  e = edge_index.shape[1]
  f32 = jnp.float32

  x2 = x.astype(f32).reshape(MR, 24)

  src = edge_index[0].astype(jnp.int32)
  dst = edge_index[1].astype(jnp.int32)
  fill = jnp.full((EPAD - e,), n, jnp.int32)
  src_c = jnp.concatenate([src, fill]).reshape(ECH, CHUNK)
  dst_c = jnp.concatenate([dst, fill]).reshape(ECH, CHUNK)

  i8 = jnp.eye(8, dtype=f32)
  kron8 = lambda wmat: jnp.kron(i8, wmat.astype(f32))
  tile8 = lambda v: jnp.tile(v.astype(f32), 8)[None, :]

  B1 = kron8(W1_lin.T)
  b1r = tile8(b1_lin)
  w1t = W1_agg.T
  K1x = kron8(w1t[:3])
  K1m = kron8(w1t[3:])
  b1ar = tile8(b1_agg)
  K2l = kron8(W2_lin.T)
  b2lr = tile8(b2_lin)
  w2t = W2_agg.T
  K2x = kron8(w2t[:16])
  K2m = kron8(w2t[16:])
  b2ar = tile8(b2_agg)
  GM = kron8(jnp.ones((F, F), f32))
  R8 = kron8(jnp.ones((1, F), f32))
  sel = jnp.zeros((F, 1), f32).at[0, 0].set(1.0)
  GS = GM @ kron8(sel)
  worep = tile8(W_out[0])
  bo8 = jnp.tile(b_out.astype(f32), 8)[None, :]

  y1 = _msg_table(x2, B1, b1r)
  s1, cnt = _edge_agg_cnt(y1.reshape(NPAD, F), src_c, dst_c)
  h1, y2 = _update1(x2, s1.reshape(2, M, 128), cnt.reshape(2, M, 8),
                    R8, K1x, K1m, b1ar, GM, K2l, b2lr)

  s2 = _edge_agg(y2.reshape(NPAD, F), src_c, dst_c)
  out8 = _update2(h1, s2.reshape(2, M, 128), cnt.reshape(2, M, 8),
                  R8, K2x, K2m, b2ar, GM, worep, GS, bo8)
  return out8.reshape(NPAD, 1)[:n]


# speedup vs baseline: 1.0652x; 1.0652x over previous
import jax
import jax.numpy as jnp
from jax import lax
from jax.experimental import pallas as pl
from jax.experimental.pallas import tpu as pltpu
from jax.experimental.pallas import tpu_sc as plsc

N_NODES = 100000
NPAD = 100352
RPT = NPAD // 16
F = 16
M = NPAD // 8
MR = N_NODES // 8

NW = 32
BIGC = 512
NGRP = 196
EPW = NGRP * BIGC
EPAD = NW * EPW

BR = 448
NG = M // BR


def _make_edge_aggregate(with_cnt):
  mesh = plsc.VectorSubcoreMesh(core_axis_name="c", subcore_axis_name="s",
                                num_cores=2, num_subcores=16)

  out_type = [jax.ShapeDtypeStruct((2, NPAD, F), jnp.float32)]
  scratch = [
      pltpu.VMEM_SHARED((NPAD, F), jnp.float32),
      pltpu.VMEM((2, BIGC), jnp.int32),
      pltpu.VMEM((2, BIGC), jnp.int32),
      pltpu.VMEM((2, BIGC, F), jnp.float32),
      pltpu.SemaphoreType.DMA,
      pltpu.SemaphoreType.DMA,
      pltpu.SemaphoreType.DMA,
      pltpu.SemaphoreType.DMA,
  ]
  if with_cnt:
    out_type.append(jax.ShapeDtypeStruct((2, NPAD), jnp.float32))
    scratch += [
        pltpu.VMEM_SHARED((NPAD,), jnp.float32),
        pltpu.VMEM((BIGC,), jnp.float32),
    ]

  def body(*refs):
    if with_cnt:
      (y_hbm, src_hbm, dst_hbm, z16_hbm, z1_hbm, out_hbm, cnt_hbm,
       acc_sh, sidx, didx, rows, gs0, gs1, ss0, ss1, cnt_sh, ones_v) = refs
    else:
      (y_hbm, src_hbm, dst_hbm, z16_hbm, out_hbm,
       acc_sh, sidx, didx, rows, gs0, gs1, ss0, ss1) = refs
    gsem = (gs0, gs1)
    ssem = (ss0, ss1)

    c = lax.axis_index("c")
    s = lax.axis_index("s")
    w = s * 2 + c
    row0 = s * RPT
    base = w * EPW

    def stage(p, g):
      pltpu.sync_copy(src_hbm.at[pl.ds(base + g * BIGC, BIGC)], sidx.at[p])
      pltpu.sync_copy(dst_hbm.at[pl.ds(base + g * BIGC, BIGC)], didx.at[p])

    def fire_g(p):
      pltpu.async_copy(y_hbm.at[sidx.at[p]], rows.at[p], gsem[p])

    def wait_g(p):
      pltpu.make_async_copy(y_hbm.at[sidx.at[p]], rows.at[p], gsem[p]).wait()

    def fire_s(p):
      pltpu.async_copy(rows.at[p], acc_sh.at[didx.at[p]], ssem[p], add=True)
      if with_cnt:
        pltpu.async_copy(ones_v, cnt_sh.at[didx.at[p]], ssem[p], add=True)

    def wait_s(p):
      pltpu.make_async_copy(rows.at[p], acc_sh.at[didx.at[p]],
                            ssem[p]).wait()
      if with_cnt:
        pltpu.make_async_copy(ones_v, cnt_sh.at[didx.at[p]],
                              ssem[p]).wait()

    pltpu.sync_copy(z16_hbm.at[pl.ds(row0, RPT)], acc_sh.at[pl.ds(row0, RPT)])
    if with_cnt:
      pltpu.sync_copy(z1_hbm.at[pl.ds(row0, RPT)], cnt_sh.at[pl.ds(row0, RPT)])
      for i in range(BIGC // 16):
        ones_v[pl.ds(i * 16, 16)] = jnp.full((16,), 1.0, jnp.float32)
    plsc.subcore_barrier()

    stage(0, 0)
    fire_g(0)

    def group2(i, carry):
      g = 2 * i
      stage(1, g + 1)
      fire_g(1)
      wait_g(0)
      fire_s(0)
      wait_s(0)

      @pl.when(i < NGRP // 2 - 1)
      def _():
        stage(0, g + 2)
        fire_g(0)

      wait_g(1)
      fire_s(1)
      wait_s(1)
      return carry

    lax.fori_loop(0, NGRP // 2, group2, 0)
    plsc.subcore_barrier()

    pltpu.sync_copy(acc_sh.at[pl.ds(row0, RPT)],
                    out_hbm.at[c, pl.ds(row0, RPT)])
    if with_cnt:
      pltpu.sync_copy(cnt_sh.at[pl.ds(row0, RPT)],
                      cnt_hbm.at[c, pl.ds(row0, RPT)])

  return pl.kernel(body,
                   out_type=tuple(out_type) if with_cnt else out_type[0],
                   mesh=mesh, scratch_types=scratch,
                   compiler_params=pltpu.CompilerParams(
                       use_tc_tiling_on_sc=False))


_edge_agg_cnt = _make_edge_aggregate(True)
_edge_agg = _make_edge_aggregate(False)


def _msg_body(x2_ref, b1_ref, bb1_ref, o_ref):
  o_ref[...] = jnp.maximum(
      jnp.dot(x2_ref[...], b1_ref[...], preferred_element_type=jnp.float32)
      + bb1_ref[...], 0.0)


def _msg_table(x2, B1, b1r):
  return pl.pallas_call(
      _msg_body,
      grid=(NG,),
      in_specs=[pl.BlockSpec((BR, 24), lambda i: (i, 0)),
                pl.BlockSpec((24, 128), lambda i: (0, 0)),
                pl.BlockSpec((1, 128), lambda i: (0, 0))],
      out_specs=pl.BlockSpec((BR, 128), lambda i: (i, 0)),
      out_shape=jax.ShapeDtypeStruct((M, 128), jnp.float32),
  )(x2, B1, b1r)


def _upd1_body(x2_ref, s_ref, c_ref, r8_ref, kx_ref, km_ref, ba_ref,
               gm_ref, k2_ref, b2_ref, h_ref, y2_ref):
  rep = jnp.dot(c_ref[0] + c_ref[1], r8_ref[...],
                preferred_element_type=jnp.float32)
  mean = (s_ref[0] + s_ref[1]) / jnp.maximum(rep, 1.0)
  h = (jnp.dot(x2_ref[...], kx_ref[...], preferred_element_type=jnp.float32)
       + jnp.dot(mean, km_ref[...], preferred_element_type=jnp.float32)
       + ba_ref[...])
  h = jnp.maximum(h, 0.0)
  n2 = jnp.dot(h * h, gm_ref[...], preferred_element_type=jnp.float32)
  h = h / jnp.maximum(jnp.sqrt(n2), 1e-12)
  h_ref[...] = h
  y2_ref[...] = jnp.maximum(
      jnp.dot(h, k2_ref[...], preferred_element_type=jnp.float32)
      + b2_ref[...], 0.0)


def _update1(x2, s1p, c8, R8, K1x, K1m, b1ar, GM, K2l, b2lr):
  full = lambda a, b: pl.BlockSpec((a, b), lambda i: (0, 0))
  return pl.pallas_call(
      _upd1_body,
      grid=(NG,),
      in_specs=[pl.BlockSpec((BR, 24), lambda i: (i, 0)),
                pl.BlockSpec((2, BR, 128), lambda i: (0, i, 0)),
                pl.BlockSpec((2, BR, 8), lambda i: (0, i, 0)),
                full(8, 128), full(24, 128), full(128, 128), full(1, 128),
                full(128, 128), full(128, 128), full(1, 128)],
      out_specs=(pl.BlockSpec((BR, 128), lambda i: (i, 0)),
                 pl.BlockSpec((BR, 128), lambda i: (i, 0))),
      out_shape=(jax.ShapeDtypeStruct((M, 128), jnp.float32),
                 jax.ShapeDtypeStruct((M, 128), jnp.float32)),
  )(x2, s1p, c8, R8, K1x, K1m, b1ar, GM, K2l, b2lr)


def _upd2_body(h1_ref, s_ref, c_ref, r8_ref, kx_ref, km_ref, ba_ref,
               gm_ref, wo_ref, gs_ref, bo_ref, o8_ref):
  rep = jnp.dot(c_ref[0] + c_ref[1], r8_ref[...],
                preferred_element_type=jnp.float32)
  mean = (s_ref[0] + s_ref[1]) / jnp.maximum(rep, 1.0)
  h = (jnp.dot(h1_ref[...], kx_ref[...], preferred_element_type=jnp.float32)
       + jnp.dot(mean, km_ref[...], preferred_element_type=jnp.float32)
       + ba_ref[...])
  h = jnp.maximum(h, 0.0)
  n2 = jnp.dot(h * h, gm_ref[...], preferred_element_type=jnp.float32)
  h = h / jnp.maximum(jnp.sqrt(n2), 1e-12)
  o8_ref[...] = jnp.dot(h * wo_ref[...], gs_ref[...],
                        preferred_element_type=jnp.float32) + bo_ref[...]


def _update2(h1, s2p, c8, R8, K2x, K2m, b2ar, GM, worep, GS, bo8):
  full = lambda a, b: pl.BlockSpec((a, b), lambda i: (0, 0))
  return pl.pallas_call(
      _upd2_body,
      grid=(NG,),
      in_specs=[pl.BlockSpec((BR, 128), lambda i: (i, 0)),
                pl.BlockSpec((2, BR, 128), lambda i: (0, i, 0)),
                pl.BlockSpec((2, BR, 8), lambda i: (0, i, 0)),
                full(8, 128), full(128, 128), full(128, 128), full(1, 128),
                full(128, 128), full(1, 128), full(128, 8), full(1, 8)],
      out_specs=pl.BlockSpec((BR, 8), lambda i: (i, 0)),
      out_shape=jax.ShapeDtypeStruct((M, 8), jnp.float32),
  )(h1, s2p, c8, R8, K2x, K2m, b2ar, GM, worep, GS, bo8)


def kernel(x, edge_index, W1_lin, b1_lin, W1_agg, b1_agg,
           W2_lin, b2_lin, W2_agg, b2_agg, W_out, b_out):
  n = x.shape[0]
  e = edge_index.shape[1]
  f32 = jnp.float32

  x2 = jnp.concatenate([x.astype(f32).reshape(MR, 24),
                        jnp.zeros((M - MR, 24), f32)])

  src = edge_index[0].astype(jnp.int32)
  dst = edge_index[1].astype(jnp.int32)
  fill = jnp.full((EPAD - e,), n, jnp.int32)
  src_c = jnp.concatenate([src, fill])
  dst_c = jnp.concatenate([dst, fill])

  i8 = jnp.eye(8, dtype=f32)
  kron8 = lambda wmat: jnp.kron(i8, wmat.astype(f32))
  tile8 = lambda v: jnp.tile(v.astype(f32), 8)[None, :]

  B1 = kron8(W1_lin.T)
  b1r = tile8(b1_lin)
  w1t = W1_agg.T
  K1x = kron8(w1t[:3])
  K1m = kron8(w1t[3:])
  b1ar = tile8(b1_agg)
  K2l = kron8(W2_lin.T)
  b2lr = tile8(b2_lin)
  w2t = W2_agg.T
  K2x = kron8(w2t[:16])
  K2m = kron8(w2t[16:])
  b2ar = tile8(b2_agg)
  GM = kron8(jnp.ones((F, F), f32))
  R8 = kron8(jnp.ones((1, F), f32))
  sel = jnp.zeros((F, 1), f32).at[0, 0].set(1.0)
  GS = GM @ kron8(sel)
  worep = tile8(W_out[0])
  bo8 = jnp.tile(b_out.astype(f32), 8)[None, :]

  z16 = jnp.zeros((NPAD, F), f32)
  z1 = jnp.zeros((NPAD,), f32)

  y1 = _msg_table(x2, B1, b1r)
  s1, cnt = _edge_agg_cnt(y1.reshape(NPAD, F), src_c, dst_c, z16, z1)
  h1, y2 = _update1(x2, s1.reshape(2, M, 128), cnt.reshape(2, M, 8),
                    R8, K1x, K1m, b1ar, GM, K2l, b2lr)

  s2 = _edge_agg(y2.reshape(NPAD, F), src_c, dst_c, z16)
  out8 = _update2(h1, s2.reshape(2, M, 128), cnt.reshape(2, M, 8),
                  R8, K2x, K2m, b2ar, GM, worep, GS, bo8)
  return out8.reshape(NPAD, 1)[:n]
